# baseline (speedup 1.0000x reference)
import dataclasses
import functools

import jax
import jax.numpy as jnp
from jax import lax
from jax.experimental import pallas as pl
from jax.experimental.pallas import tpu as pltpu
from jax.experimental.pallas import tpu_sc as plsc

_B, _S, _D = 2, 2048, 1024
_H = 2048
_MH = 64
_E = 8
_V = 32000
_N = _B * _S

_BN = 256

_NC, _NS, _LANES = 2, 16, 16
_NW = _NC * _NS
_PER_W = _N // _NW


def _gains_sc_kernel(table_hbm, ids_hbm, out_hbm, table_v, idx_v, vals_v, sem):
    wid = lax.axis_index("s") * _NC + lax.axis_index("c")
    base = wid * _PER_W
    pltpu.async_copy(table_hbm, table_v, sem).wait()
    pltpu.sync_copy(ids_hbm.at[pl.ds(base, _PER_W)], idx_v)

    @pl.loop(0, _PER_W, step=_LANES)
    def _(i):
        idx = idx_v[pl.ds(i, _LANES)]
        v = plsc.load_gather(table_v, [idx])
        vals_v[pl.ds(i, _LANES)] = 1.0 / (1.0 + jnp.exp(-v)) + 0.5

    pltpu.sync_copy(vals_v, out_hbm.at[pl.ds(base, _PER_W)])


def _gains_sc(prosody_table, ids_flat):
    mesh = plsc.VectorSubcoreMesh(core_axis_name="c", subcore_axis_name="s")
    cp = pltpu.CompilerParams()
    if "needs_layout_passes" in pltpu.CompilerParams.__dataclass_fields__:
        cp = dataclasses.replace(cp, needs_layout_passes=False)
    k = pl.kernel(
        _gains_sc_kernel,
        out_type=jax.ShapeDtypeStruct((_N,), jnp.float32),
        mesh=mesh,
        scratch_types=[
            pltpu.VMEM((_V,), jnp.float32),
            pltpu.VMEM((_PER_W,), jnp.int32),
            pltpu.VMEM((_PER_W,), jnp.float32),
            pltpu.SemaphoreType.DMA,
        ],
        compiler_params=cp,
    )
    return k(prosody_table, ids_flat)


def _tc_body(x_ref, g_ref, wenc_ref, benc_ref, ws2c_ref, bs2c_ref,
             wr1_ref, br1_ref, wr2_ref, br2_ref,
             we1_ref, be1_ref, we2_ref, be2_ref,
             wc2s_ref, bc2s_ref, wdec_ref, bdec_ref,
             lng_ref, lnb_ref, out_ref):
    f32 = jnp.float32
    g = g_ref[...]
    x = (x_ref[...] * g).astype(jnp.bfloat16)

    a = jnp.dot(x, wenc_ref[...], preferred_element_type=f32) + benc_ref[...]
    a = jax.nn.sigmoid(4.0 * a)

    cont = (jnp.dot(a.astype(jnp.bfloat16), ws2c_ref[...],
                    preferred_element_type=f32) + bs2c_ref[...])

    h = jnp.tanh(jnp.dot(cont, wr1_ref[...], preferred_element_type=f32)
                 + br1_ref[...])
    logits = (jnp.dot(h, wr2_ref[...], preferred_element_type=f32)
              + br2_ref[...]) * g
    m = jnp.max(logits, axis=-1, keepdims=True)
    p = jnp.exp(logits - m)
    p = p / jnp.sum(p, axis=-1, keepdims=True)

    eidx = lax.broadcasted_iota(jnp.int32, (_BN, _E), 1)
    m1 = jnp.max(p, axis=-1, keepdims=True)
    i1 = jnp.min(jnp.where(p >= m1, eidx, _E), axis=-1, keepdims=True)
    oh1 = eidx == i1
    pm = jnp.where(oh1, -1.0, p)
    m2 = jnp.max(pm, axis=-1, keepdims=True)
    i2 = jnp.min(jnp.where(pm >= m2, eidx, _E), axis=-1, keepdims=True)
    oh2 = eidx == i2
    denom = m1 + m2 + 1e-9
    gate = (jnp.where(oh1, m1, 0.0) + jnp.where(oh2, m2, 0.0)) / denom

    h1 = jax.nn.sigmoid(
        4.0 * (jnp.dot(cont.astype(jnp.bfloat16), we1_ref[...],
                       preferred_element_type=f32) + be1_ref[...]))
    h1 = (h1.reshape(_BN, _E, _H // 2) * gate[:, :, None]).reshape(
        _BN, _E * (_H // 2))
    eo = (jnp.dot(h1.astype(jnp.bfloat16), we2_ref[...],
                  preferred_element_type=f32)
          + jnp.dot(gate, be2_ref[...], preferred_element_type=f32))

    rates = jax.nn.sigmoid(
        jnp.dot(eo.astype(jnp.bfloat16), wc2s_ref[...],
                preferred_element_type=f32) + bc2s_ref[...])
    avg = (rates * g).astype(jnp.bfloat16)

    z = jnp.dot(avg, wdec_ref[...], preferred_element_type=f32) + bdec_ref[...]
    dec = jax.nn.sigmoid(4.0 * z)

    mu = jnp.mean(dec, axis=-1, keepdims=True)
    var = jnp.mean((dec - mu) ** 2, axis=-1, keepdims=True)
    out_ref[...] = ((dec - mu) / jnp.sqrt(var + 1e-5)) * lng_ref[...] + lnb_ref[...]


def _full(shape):
    nd = len(shape)
    return pl.BlockSpec(shape, lambda i, _nd=nd: (0,) * _nd)


def _tc_call(x, gains, wenc, benc, ws2c, bs2c, wr1, br1, wr2, br2,
             we1, be1, we2, be2, wc2s, bc2s, wdec, bdec, lng, lnb):
    grid = (_N // _BN,)
    in_specs = [
        pl.BlockSpec((_BN, _D), lambda i: (i, 0)),
        pl.BlockSpec((_BN, 1), lambda i: (i, 0)),
        _full(wenc.shape), _full(benc.shape),
        _full(ws2c.shape), _full(bs2c.shape),
        _full(wr1.shape), _full(br1.shape),
        _full(wr2.shape), _full(br2.shape),
        _full(we1.shape), _full(be1.shape),
        _full(we2.shape), _full(be2.shape),
        _full(wc2s.shape), _full(bc2s.shape),
        _full(wdec.shape), _full(bdec.shape),
        _full(lng.shape), _full(lnb.shape),
    ]
    return pl.pallas_call(
        _tc_body,
        grid=grid,
        in_specs=in_specs,
        out_specs=pl.BlockSpec((_BN, _D), lambda i: (i, 0)),
        out_shape=jax.ShapeDtypeStruct((_N, _D), jnp.float32),
        compiler_params=pltpu.CompilerParams(
            dimension_semantics=("parallel",)),
    )(x, gains, wenc, benc, ws2c, bs2c, wr1, br1, wr2, br2,
      we1, be1, we2, be2, wc2s, bc2s, wdec, bdec, lng, lnb)


def kernel(inputs_embeds, input_ids, prosody_table, W_enc, b_enc, W_s2c, b_s2c,
           W_r1, b_r1, W_r2, b_r2, W_e1, b_e1, W_e2, b_e2,
           W_c2s, b_c2s, W_dec, b_dec, ln_g, ln_b):
    bf16 = jnp.bfloat16
    gains = _gains_sc(prosody_table, input_ids.reshape(_N))
    out = _tc_call(
        inputs_embeds.reshape(_N, _D),
        gains.reshape(_N, 1),
        W_enc.astype(bf16), b_enc.reshape(1, _H),
        W_s2c.astype(bf16), b_s2c.reshape(1, _MH),
        W_r1, b_r1.reshape(1, 64),
        W_r2, b_r2.reshape(1, _E),
        W_e1.transpose(1, 0, 2).reshape(_MH, _E * (_H // 2)).astype(bf16),
        b_e1.reshape(1, _E * (_H // 2)),
        W_e2.reshape(_E * (_H // 2), _MH).astype(bf16),
        b_e2,
        W_c2s.astype(bf16), b_c2s.reshape(1, _H),
        W_dec.astype(bf16), b_dec.reshape(1, _D),
        ln_g.reshape(1, _D), ln_b.reshape(1, _D),
    )
    return out.reshape(_B, _S, _D)


# speedup vs baseline: 1.0529x; 1.0529x over previous
import dataclasses
import functools

import jax
import jax.numpy as jnp
from jax import lax
from jax.experimental import pallas as pl
from jax.experimental.pallas import tpu as pltpu
from jax.experimental.pallas import tpu_sc as plsc

_B, _S, _D = 2, 2048, 1024
_H = 2048
_MH = 64
_E = 8
_V = 32000
_N = _B * _S
_H2 = _H // 2

_BN = 256

_NC, _NS, _LANES = 2, 16, 16
_NW = _NC * _NS
_PER_W = _N // _NW


def _gains_sc_kernel(table_hbm, ids_hbm, out_hbm, table_v, idx_v, vals_v, sem):
    wid = lax.axis_index("s") * _NC + lax.axis_index("c")
    base = wid * _PER_W
    pltpu.async_copy(table_hbm, table_v, sem).wait()
    pltpu.sync_copy(ids_hbm.at[pl.ds(base, _PER_W)], idx_v)

    @pl.loop(0, _PER_W, step=_LANES)
    def _(i):
        idx = idx_v[pl.ds(i, _LANES)]
        v = plsc.load_gather(table_v, [idx])
        vals_v[pl.ds(i, _LANES)] = 1.0 / (1.0 + jnp.exp(-v)) + 0.5

    pltpu.sync_copy(vals_v, out_hbm.at[pl.ds(base, _PER_W)])


def _gains_sc(prosody_table, ids_flat):
    mesh = plsc.VectorSubcoreMesh(core_axis_name="c", subcore_axis_name="s")
    cp = pltpu.CompilerParams()
    if "needs_layout_passes" in pltpu.CompilerParams.__dataclass_fields__:
        cp = dataclasses.replace(cp, needs_layout_passes=False)
    k = pl.kernel(
        _gains_sc_kernel,
        out_type=jax.ShapeDtypeStruct((_N,), jnp.float32),
        mesh=mesh,
        scratch_types=[
            pltpu.VMEM((_V,), jnp.float32),
            pltpu.VMEM((_PER_W,), jnp.int32),
            pltpu.VMEM((_PER_W,), jnp.float32),
            pltpu.SemaphoreType.DMA,
        ],
        compiler_params=cp,
    )
    return k(prosody_table, ids_flat)


def _sig2(z):
    return 0.5 + 0.5 * jnp.tanh(z)


def _tc_body(x_ref, g_ref, wencT_ref, benc2_ref, ws2cT_ref, bs2c_ref,
             wr1T_ref, br1_ref, wr2T_ref, br2_ref,
             we1T_ref, we2T_ref, be2T_ref,
             wc2sT_ref, wdecT_ref, bdec2_ref,
             lng_ref, lnb_ref, out_ref):
    f32 = jnp.float32
    g = g_ref[...]
    x = x_ref[...].astype(jnp.bfloat16)

    z1 = lax.dot_general(wencT_ref[...], x, (((1,), (1,)), ((), ())),
                         preferred_element_type=f32)
    a = _sig2(z1 * g + benc2_ref[...])

    cont = (jnp.dot(ws2cT_ref[...], a.astype(jnp.bfloat16),
                    preferred_element_type=f32) + bs2c_ref[...])

    h = jnp.tanh(jnp.dot(wr1T_ref[...], cont, preferred_element_type=f32)
                 + br1_ref[...])
    logits = (jnp.dot(wr2T_ref[...], h, preferred_element_type=f32)
              + br2_ref[...]) * g
    m = jnp.max(logits, axis=0, keepdims=True)
    p = jnp.exp(logits - m)
    p = p / jnp.sum(p, axis=0, keepdims=True)

    eidx = lax.broadcasted_iota(jnp.int32, (_E, _BN), 0)
    m1 = jnp.max(p, axis=0, keepdims=True)
    i1 = jnp.min(jnp.where(p >= m1, eidx, _E), axis=0, keepdims=True)
    oh1 = eidx == i1
    pm = jnp.where(oh1, -1.0, p)
    m2 = jnp.max(pm, axis=0, keepdims=True)
    i2 = jnp.min(jnp.where(pm >= m2, eidx, _E), axis=0, keepdims=True)
    oh2 = eidx == i2
    denom = m1 + m2 + 1e-9
    gate = (jnp.where(oh1, m1, 0.0) + jnp.where(oh2, m2, 0.0)) / denom

    ones_row = jnp.ones((1, _BN), f32)
    cont_ext = jnp.concatenate([cont, ones_row], axis=0).astype(jnp.bfloat16)
    h1 = _sig2(jnp.dot(we1T_ref[...], cont_ext,
                       preferred_element_type=f32))
    eo = jnp.dot(be2T_ref[...], gate, preferred_element_type=f32)
    for e in range(_E):
        blk = (h1[e * _H2:(e + 1) * _H2] * gate[e:e + 1]).astype(jnp.bfloat16)
        eo = eo + jnp.dot(we2T_ref[e], blk, preferred_element_type=f32)

    eo_ext = jnp.concatenate([eo, ones_row], axis=0).astype(jnp.bfloat16)
    rates = _sig2(jnp.dot(wc2sT_ref[...], eo_ext,
                          preferred_element_type=f32))
    avg = (rates * g).astype(jnp.bfloat16)

    z = (jnp.dot(wdecT_ref[...], avg, preferred_element_type=f32)
         + bdec2_ref[...])
    dec = _sig2(z)

    mu = jnp.mean(dec, axis=0, keepdims=True)
    var = jnp.mean((dec - mu) ** 2, axis=0, keepdims=True)
    outT = ((dec - mu) * lax.rsqrt(var + 1e-5)) * lng_ref[...] + lnb_ref[...]
    out_ref[...] = outT.T


def _full(shape):
    nd = len(shape)
    return pl.BlockSpec(shape, lambda i, _nd=nd: (0,) * _nd)


def _tc_call(x, gains_row, *weights):
    in_specs = [
        pl.BlockSpec((_BN, _D), lambda i: (i, 0)),
        pl.BlockSpec((1, _BN), lambda i: (0, i)),
    ] + [_full(w.shape) for w in weights]
    return pl.pallas_call(
        _tc_body,
        grid=(_N // _BN,),
        in_specs=in_specs,
        out_specs=pl.BlockSpec((_BN, _D), lambda i: (i, 0)),
        out_shape=jax.ShapeDtypeStruct((_N, _D), jnp.float32),
        compiler_params=pltpu.CompilerParams(
            dimension_semantics=("parallel",)),
    )(x, gains_row, *weights)


def kernel(inputs_embeds, input_ids, prosody_table, W_enc, b_enc, W_s2c, b_s2c,
           W_r1, b_r1, W_r2, b_r2, W_e1, b_e1, W_e2, b_e2,
           W_c2s, b_c2s, W_dec, b_dec, ln_g, ln_b):
    bf16 = jnp.bfloat16
    f32 = jnp.float32
    gains = _gains_sc(prosody_table, input_ids.reshape(_N))

    we1T = jnp.concatenate(
        [(2.0 * W_e1).transpose(0, 2, 1).reshape(_E * _H2, _MH),
         (2.0 * b_e1).reshape(_E * _H2, 1)], axis=1).astype(bf16)
    wc2sT = jnp.concatenate(
        [(0.5 * W_c2s).T, (0.5 * b_c2s).reshape(_H, 1)], axis=1).astype(bf16)

    weights = (
        (2.0 * W_enc).T.astype(bf16),
        jnp.broadcast_to((2.0 * b_enc)[:, None], (_H, _BN)).astype(f32),
        W_s2c.T.astype(bf16),
        b_s2c.reshape(_MH, 1),
        W_r1.T, b_r1.reshape(64, 1),
        W_r2.T, b_r2.reshape(_E, 1),
        we1T,
        W_e2.transpose(0, 2, 1).astype(bf16),
        b_e2.T,
        wc2sT,
        (2.0 * W_dec).T.astype(bf16),
        jnp.broadcast_to((2.0 * b_dec)[:, None], (_D, _BN)).astype(f32),
        jnp.broadcast_to(ln_g[:, None], (_D, _BN)).astype(f32),
        jnp.broadcast_to(ln_b[:, None], (_D, _BN)).astype(f32),
    )
    out = _tc_call(inputs_embeds.reshape(_N, _D), gains.reshape(1, _N),
                   *weights)
    return out.reshape(_B, _S, _D)


# speedup vs baseline: 1.1173x; 1.0612x over previous
import dataclasses
import functools

import jax
import jax.numpy as jnp
from jax import lax
from jax.experimental import pallas as pl
from jax.experimental.pallas import tpu as pltpu
from jax.experimental.pallas import tpu_sc as plsc

_B, _S, _D = 2, 2048, 1024
_H = 2048
_MH = 64
_E = 8
_V = 32000
_N = _B * _S
_H2 = _H // 2

_BN = 256

_NC, _NS, _LANES = 2, 16, 16
_NW = _NC * _NS
_PER_W = _N // _NW


def _gains_sc_kernel(table_hbm, ids_hbm, out_hbm, table_v, idx_v, vals_v, sem):
    wid = lax.axis_index("s") * _NC + lax.axis_index("c")
    base = wid * _PER_W
    pltpu.async_copy(table_hbm, table_v, sem).wait()
    pltpu.sync_copy(ids_hbm.at[pl.ds(base, _PER_W)], idx_v)

    @pl.loop(0, _PER_W, step=_LANES)
    def _(i):
        idx = idx_v[pl.ds(i, _LANES)]
        v = plsc.load_gather(table_v, [idx])
        vals_v[pl.ds(i, _LANES)] = 1.0 / (1.0 + jnp.exp(-v)) + 0.5

    pltpu.sync_copy(vals_v, out_hbm.at[pl.ds(base, _PER_W)])


def _gains_sc(prosody_table, ids_flat):
    mesh = plsc.VectorSubcoreMesh(core_axis_name="c", subcore_axis_name="s")
    cp = pltpu.CompilerParams()
    if "needs_layout_passes" in pltpu.CompilerParams.__dataclass_fields__:
        cp = dataclasses.replace(cp, needs_layout_passes=False)
    k = pl.kernel(
        _gains_sc_kernel,
        out_type=jax.ShapeDtypeStruct((_N,), jnp.float32),
        mesh=mesh,
        scratch_types=[
            pltpu.VMEM((_V,), jnp.float32),
            pltpu.VMEM((_PER_W,), jnp.int32),
            pltpu.VMEM((_PER_W,), jnp.float32),
            pltpu.SemaphoreType.DMA,
        ],
        compiler_params=cp,
    )
    return k(prosody_table, ids_flat)


def _sig2(z):
    return 0.5 + 0.5 * jnp.tanh(z)


def _dot0(w, act):
    return lax.dot_general(w, act, (((0,), (0,)), ((), ())),
                           preferred_element_type=jnp.float32)


def _tc_body(x_ref, g_ref, wenc_ref, benc2_ref, ws2c_ref, bs2c_ref,
             wr1_ref, br1_ref, wr2_ref, br2_ref,
             we1x_ref, we2_ref, be2_ref,
             wc2sx_ref, wdec_ref, bdec2_ref,
             lng_ref, lnb_ref, out_ref):
    f32 = jnp.float32
    g = g_ref[...]
    g2 = g + g
    x = x_ref[...].astype(jnp.bfloat16)

    z1 = lax.dot_general(wenc_ref[...], x, (((0,), (1,)), ((), ())),
                         preferred_element_type=f32)
    a = _sig2(z1 * g2 + benc2_ref[...])

    cont = (_dot0(ws2c_ref[...], a.astype(jnp.bfloat16))
            + bs2c_ref[...])

    h = jnp.tanh(_dot0(wr1_ref[...], cont) + br1_ref[...])
    logits = (_dot0(wr2_ref[...], h) + br2_ref[...]) * g
    m = jnp.max(logits, axis=0, keepdims=True)
    p = jnp.exp(logits - m)
    p = p / jnp.sum(p, axis=0, keepdims=True)

    eidx = lax.broadcasted_iota(jnp.int32, (_E, _BN), 0)
    m1 = jnp.max(p, axis=0, keepdims=True)
    i1 = jnp.min(jnp.where(p >= m1, eidx, _E), axis=0, keepdims=True)
    oh1 = eidx == i1
    pm = jnp.where(oh1, -1.0, p)
    m2 = jnp.max(pm, axis=0, keepdims=True)
    i2 = jnp.min(jnp.where(pm >= m2, eidx, _E), axis=0, keepdims=True)
    oh2 = eidx == i2
    denom = m1 + m2 + 1e-9
    gate = (jnp.where(oh1, m1, 0.0) + jnp.where(oh2, m2, 0.0)) / denom

    ones_row = jnp.ones((1, _BN), f32)
    cont_ext = jnp.concatenate([cont, ones_row], axis=0).astype(jnp.bfloat16)
    h1 = _sig2(_dot0(we1x_ref[...], cont_ext))
    eo = _dot0(be2_ref[...], gate)
    for e in range(_E):
        blk = (h1[e * _H2:(e + 1) * _H2] * gate[e:e + 1]).astype(jnp.bfloat16)
        eo = eo + _dot0(we2_ref[e], blk)

    eo_ext = jnp.concatenate([eo, ones_row], axis=0).astype(jnp.bfloat16)
    rates = _sig2(_dot0(wc2sx_ref[...], eo_ext))
    avg = (rates * g2).astype(jnp.bfloat16)

    z = _dot0(wdec_ref[...], avg) + bdec2_ref[...]
    dec = _sig2(z)

    mu = jnp.mean(dec, axis=0, keepdims=True)
    var = jnp.mean((dec - mu) ** 2, axis=0, keepdims=True)
    outT = ((dec - mu) * lax.rsqrt(var + 1e-5)) * lng_ref[...] + lnb_ref[...]
    out_ref[...] = outT.T


def _full(shape):
    nd = len(shape)
    return pl.BlockSpec(shape, lambda i, _nd=nd: (0,) * _nd)


def _tc_call(x, gains_row, *weights):
    in_specs = [
        pl.BlockSpec((_BN, _D), lambda i: (i, 0)),
        pl.BlockSpec((1, _BN), lambda i: (0, i)),
    ] + [_full(w.shape) for w in weights]
    return pl.pallas_call(
        _tc_body,
        grid=(_N // _BN,),
        in_specs=in_specs,
        out_specs=pl.BlockSpec((_BN, _D), lambda i: (i, 0)),
        out_shape=jax.ShapeDtypeStruct((_N, _D), jnp.float32),
        compiler_params=pltpu.CompilerParams(
            dimension_semantics=("parallel",)),
    )(x, gains_row, *weights)


def kernel(inputs_embeds, input_ids, prosody_table, W_enc, b_enc, W_s2c, b_s2c,
           W_r1, b_r1, W_r2, b_r2, W_e1, b_e1, W_e2, b_e2,
           W_c2s, b_c2s, W_dec, b_dec, ln_g, ln_b):
    bf16 = jnp.bfloat16
    f32 = jnp.float32
    gains = _gains_sc(prosody_table, input_ids.reshape(_N))

    we1x = jnp.concatenate(
        [(2.0 * W_e1).transpose(1, 0, 2).reshape(_MH, _E * _H2),
         (2.0 * b_e1).reshape(1, _E * _H2)], axis=0).astype(bf16)
    wc2sx = jnp.concatenate(
        [0.5 * W_c2s, (0.5 * b_c2s).reshape(1, _H)], axis=0).astype(bf16)

    weights = (
        W_enc.astype(bf16),
        jnp.broadcast_to((2.0 * b_enc)[:, None], (_H, _BN)).astype(f32),
        W_s2c.astype(bf16),
        b_s2c.reshape(_MH, 1),
        W_r1, b_r1.reshape(64, 1),
        W_r2, b_r2.reshape(_E, 1),
        we1x,
        W_e2.astype(bf16),
        b_e2,
        wc2sx,
        W_dec.astype(bf16),
        jnp.broadcast_to((2.0 * b_dec)[:, None], (_D, _BN)).astype(f32),
        jnp.broadcast_to(ln_g[:, None], (_D, _BN)).astype(f32),
        jnp.broadcast_to(ln_b[:, None], (_D, _BN)).astype(f32),
    )
    out = _tc_call(inputs_embeds.reshape(_N, _D), gains.reshape(1, _N),
                   *weights)
    return out.reshape(_B, _S, _D)


# speedup vs baseline: 1.3582x; 1.2156x over previous
import dataclasses
import functools

import jax
import jax.numpy as jnp
from jax import lax
from jax.experimental import pallas as pl
from jax.experimental.pallas import tpu as pltpu
from jax.experimental.pallas import tpu_sc as plsc

_B, _S, _D = 2, 2048, 1024
_H = 2048
_MH = 64
_E = 8
_V = 32000
_N = _B * _S
_H2 = _H // 2

_BN = 256
_G = _N // _BN

_NC, _NS, _LANES = 2, 16, 16
_NW = _NC * _NS
_PER_W = _N // _NW


def _gains_sc_kernel(table_hbm, ids_hbm, out_hbm, table_v, idx_v, vals_v, sem):
    wid = lax.axis_index("s") * _NC + lax.axis_index("c")
    base = wid * _PER_W
    pltpu.async_copy(table_hbm, table_v, sem).wait()
    pltpu.sync_copy(ids_hbm.at[pl.ds(base, _PER_W)], idx_v)

    @pl.loop(0, _PER_W, step=_LANES)
    def _(i):
        idx = idx_v[pl.ds(i, _LANES)]
        v = plsc.load_gather(table_v, [idx])
        vals_v[pl.ds(i, _LANES)] = 1.0 / (1.0 + jnp.exp(-v)) + 0.5

    pltpu.sync_copy(vals_v, out_hbm.at[pl.ds(base, _PER_W)])


def _gains_sc(prosody_table, ids_flat):
    mesh = plsc.VectorSubcoreMesh(core_axis_name="c", subcore_axis_name="s")
    cp = pltpu.CompilerParams()
    if "needs_layout_passes" in pltpu.CompilerParams.__dataclass_fields__:
        cp = dataclasses.replace(cp, needs_layout_passes=False)
    k = pl.kernel(
        _gains_sc_kernel,
        out_type=jax.ShapeDtypeStruct((_N,), jnp.float32),
        mesh=mesh,
        scratch_types=[
            pltpu.VMEM((_V,), jnp.float32),
            pltpu.VMEM((_PER_W,), jnp.int32),
            pltpu.VMEM((_PER_W,), jnp.float32),
            pltpu.SemaphoreType.DMA,
        ],
        compiler_params=cp,
    )
    return k(prosody_table, ids_flat)


def _sig2(z):
    return 0.5 + 0.5 * jnp.tanh(z)


def _dot0(w, act):
    return lax.dot_general(w, act, (((0,), (0,)), ((), ())),
                           preferred_element_type=jnp.float32)


def _tc_body(x_ref, g1_ref, g2_ref, wenc_ref, benc2_ref, ws2c_ref, bs2c_ref,
             wr1_ref, br1_ref, wr2_ref, br2_ref,
             we1x_ref, we2_ref, be2_ref,
             wc2sx_ref, wdec_ref, bdec2_ref,
             lng_ref, lnb_ref, out_ref, cont_scr):
    f32 = jnp.float32
    i = pl.program_id(0)
    par = lax.rem(i, 2)

    @pl.when(i > 0)
    def _back():
        g = g2_ref[...]
        g2 = g + g
        cont = cont_scr[1 - par]

        h = jnp.tanh(_dot0(wr1_ref[...], cont) + br1_ref[...])
        logits = (_dot0(wr2_ref[...], h) + br2_ref[...]) * g
        m = jnp.max(logits, axis=0, keepdims=True)
        p = jnp.exp(logits - m)
        p = p / jnp.sum(p, axis=0, keepdims=True)

        eidx = lax.broadcasted_iota(jnp.int32, (_E, _BN), 0)
        m1 = jnp.max(p, axis=0, keepdims=True)
        i1 = jnp.min(jnp.where(p >= m1, eidx, _E), axis=0, keepdims=True)
        oh1 = eidx == i1
        pm = jnp.where(oh1, -1.0, p)
        m2 = jnp.max(pm, axis=0, keepdims=True)
        i2 = jnp.min(jnp.where(pm >= m2, eidx, _E), axis=0, keepdims=True)
        oh2 = eidx == i2
        denom = m1 + m2 + 1e-9
        gate = (jnp.where(oh1, m1, 0.0) + jnp.where(oh2, m2, 0.0)) / denom

        ones_row = jnp.ones((1, _BN), f32)
        cont_ext = jnp.concatenate([cont, ones_row], 0).astype(jnp.bfloat16)
        h1 = _sig2(_dot0(we1x_ref[...], cont_ext))
        eo = _dot0(be2_ref[...], gate)
        for e in range(_E):
            blk = (h1[e * _H2:(e + 1) * _H2]
                   * gate[e:e + 1]).astype(jnp.bfloat16)
            eo = eo + _dot0(we2_ref[e], blk)

        eo_ext = jnp.concatenate([eo, ones_row], 0).astype(jnp.bfloat16)
        rates = _sig2(_dot0(wc2sx_ref[...], eo_ext))
        avg = (rates * g2).astype(jnp.bfloat16)

        z = lax.dot_general(avg, wdec_ref[...], (((0,), (0,)), ((), ())),
                            preferred_element_type=f32) + bdec2_ref[...]
        dec = _sig2(z)

        mu = jnp.mean(dec, axis=1, keepdims=True)
        var = jnp.mean((dec - mu) ** 2, axis=1, keepdims=True)
        out_ref[...] = (((dec - mu) * lax.rsqrt(var + 1e-5)) * lng_ref[...]
                        + lnb_ref[...])

    @pl.when(i < _G)
    def _front():
        g = g1_ref[...]
        g2 = g + g
        x = x_ref[...].astype(jnp.bfloat16)
        z1 = lax.dot_general(wenc_ref[...], x, (((0,), (1,)), ((), ())),
                             preferred_element_type=f32)
        a = _sig2(z1 * g2 + benc2_ref[...])
        cont_scr[par] = (_dot0(ws2c_ref[...], a.astype(jnp.bfloat16))
                         + bs2c_ref[...])


def _full(shape):
    nd = len(shape)
    return pl.BlockSpec(shape, lambda i, _nd=nd: (0,) * _nd)


def _tc_call(x, gains_row, *weights):
    in_specs = [
        pl.BlockSpec((_BN, _D), lambda i: (jnp.minimum(i, _G - 1), 0)),
        pl.BlockSpec((1, _BN), lambda i: (0, jnp.minimum(i, _G - 1))),
        pl.BlockSpec((1, _BN), lambda i: (0, jnp.maximum(i - 1, 0))),
    ] + [_full(w.shape) for w in weights]
    return pl.pallas_call(
        _tc_body,
        grid=(_G + 1,),
        in_specs=in_specs,
        out_specs=pl.BlockSpec((_BN, _D),
                               lambda i: (jnp.maximum(i - 1, 0), 0)),
        out_shape=jax.ShapeDtypeStruct((_N, _D), jnp.float32),
        scratch_shapes=[pltpu.VMEM((2, _MH, _BN), jnp.float32)],
        compiler_params=pltpu.CompilerParams(
            dimension_semantics=("arbitrary",)),
    )(x, gains_row, gains_row, *weights)


def kernel(inputs_embeds, input_ids, prosody_table, W_enc, b_enc, W_s2c, b_s2c,
           W_r1, b_r1, W_r2, b_r2, W_e1, b_e1, W_e2, b_e2,
           W_c2s, b_c2s, W_dec, b_dec, ln_g, ln_b):
    bf16 = jnp.bfloat16
    f32 = jnp.float32
    gains = _gains_sc(prosody_table, input_ids.reshape(_N))

    we1x = jnp.concatenate(
        [(2.0 * W_e1).transpose(1, 0, 2).reshape(_MH, _E * _H2),
         (2.0 * b_e1).reshape(1, _E * _H2)], axis=0).astype(bf16)
    wc2sx = jnp.concatenate(
        [0.5 * W_c2s, (0.5 * b_c2s).reshape(1, _H)], axis=0).astype(bf16)

    weights = (
        W_enc.astype(bf16),
        jnp.broadcast_to((2.0 * b_enc)[:, None], (_H, _BN)).astype(f32),
        W_s2c.astype(bf16),
        b_s2c.reshape(_MH, 1),
        W_r1, b_r1.reshape(64, 1),
        W_r2, b_r2.reshape(_E, 1),
        we1x,
        W_e2.astype(bf16),
        b_e2,
        wc2sx,
        W_dec.astype(bf16),
        (2.0 * b_dec).reshape(1, _D),
        ln_g.reshape(1, _D),
        ln_b.reshape(1, _D),
    )
    out = _tc_call(inputs_embeds.reshape(_N, _D), gains.reshape(1, _N),
                   *weights)
    return out.reshape(_B, _S, _D)
